# Initial kernel scaffold; baseline (speedup 1.0000x reference)
import jax
import jax.numpy as jnp
from jax.experimental import pallas as pl


def kernel(x, edge_index, W1, b1, W2, b2):
    raise NotImplementedError("write your pallas kernel here")


# speedup vs baseline: 20.9470x; 20.9470x over previous
import functools

import jax
import jax.numpy as jnp
from jax import lax
from jax.experimental import pallas as pl
from jax.experimental.pallas import tpu as pltpu
from jax.experimental.pallas import tpu_sc as plsc

NC = 2
NS = 16
NW = NC * NS
CH = 128
L = 16


def _sc_mesh():
    return plsc.VectorSubcoreMesh(core_axis_name="c", subcore_axis_name="s")


_SC_PARAMS = pltpu.CompilerParams(use_tc_tiling_on_sc=False)


def _deg_kernel(n_acc, chunks):
    zrows = n_acc // NS

    @functools.partial(
        pl.kernel,
        out_type=jax.ShapeDtypeStruct((NC, n_acc), jnp.float32),
        mesh=_sc_mesh(),
        compiler_params=_SC_PARAMS,
        scratch_types=[
            pltpu.VMEM((CH,), jnp.int32),
            pltpu.VMEM((CH,), jnp.float32),
            pltpu.VMEM((zrows,), jnp.float32),
            pltpu.VMEM_SHARED((n_acc,), jnp.float32),
        ],
    )
    def deg_kernel(dst_hbm, out_hbm, didx, ones_v, zbuf, acc):
        c = lax.axis_index("c")
        s = lax.axis_index("s")
        wid = c * NS + s

        def _fill(i, _):
            zbuf[pl.ds(i * L, L)] = jnp.zeros((L,), jnp.float32)
            ones_v[pl.ds(lax.rem(i, CH // L) * L, L)] = jnp.ones((L,), jnp.float32)
            return 0

        lax.fori_loop(0, zrows // L, _fill, 0)
        pltpu.sync_copy(zbuf, acc.at[pl.ds(s * zrows, zrows)])
        plsc.subcore_barrier()

        def _edge_chunk(i, _):
            pltpu.sync_copy(dst_hbm.at[wid, i], didx)
            pltpu.sync_copy(ones_v, acc.at[didx], add=True)
            return 0

        lax.fori_loop(0, chunks, _edge_chunk, 0)
        plsc.subcore_barrier()
        pltpu.sync_copy(acc.at[pl.ds(s * zrows, zrows)],
                        out_hbm.at[c, pl.ds(s * zrows, zrows)])

    return deg_kernel


def _agg_kernel(n_acc, chunks, d):
    zrows = n_acc // NS

    @functools.partial(
        pl.kernel,
        out_type=jax.ShapeDtypeStruct((NC, n_acc, d), jnp.float32),
        mesh=_sc_mesh(),
        compiler_params=_SC_PARAMS,
        scratch_types=[
            pltpu.VMEM((CH,), jnp.int32),
            pltpu.VMEM((CH,), jnp.int32),
            pltpu.VMEM((CH, d), jnp.float32),
            pltpu.VMEM((zrows, d), jnp.float32),
            pltpu.VMEM_SHARED((n_acc, d), jnp.float32),
            pltpu.SemaphoreType.DMA,
        ],
    )
    def agg_kernel(src_hbm, dst_hbm, table_hbm, out_hbm,
                   sidx, didx, rows, zbuf, acc, sem):
        c = lax.axis_index("c")
        s = lax.axis_index("s")
        wid = c * NS + s

        def _fill(i, _):
            zbuf[i, :] = jnp.zeros((L,), jnp.float32)
            return 0

        lax.fori_loop(0, zrows, _fill, 0)
        pltpu.sync_copy(zbuf, acc.at[pl.ds(s * zrows, zrows)])
        plsc.subcore_barrier()

        def _edge_chunk(i, _):
            pltpu.sync_copy(src_hbm.at[wid, i], sidx)
            pltpu.sync_copy(dst_hbm.at[wid, i], didx)
            pltpu.async_copy(table_hbm.at[sidx], rows, sem).wait()
            pltpu.sync_copy(rows, acc.at[didx], add=True)
            return 0

        lax.fori_loop(0, chunks, _edge_chunk, 0)
        plsc.subcore_barrier()
        pltpu.sync_copy(acc.at[pl.ds(s * zrows, zrows)],
                        out_hbm.at[c, pl.ds(s * zrows, zrows)])

    return agg_kernel


def _tc_pre(x, w1, p0, p1):
    n, d_hid = x.shape[0], w1.shape[1]

    def body(x_ref, w1_ref, p0_ref, p1_ref, dinv_ref, hpre_ref, self1_ref):
        dinv = lax.rsqrt(1.0 + p0_ref[...] + p1_ref[...])
        h1 = jnp.dot(x_ref[...], w1_ref[...], preferred_element_type=jnp.float32)
        dinv_ref[...] = dinv
        hpre_ref[...] = h1 * dinv
        self1_ref[...] = h1 * (dinv * dinv)

    return pl.pallas_call(
        body,
        out_shape=(
            jax.ShapeDtypeStruct((n, 1), jnp.float32),
            jax.ShapeDtypeStruct((n, d_hid), jnp.float32),
            jax.ShapeDtypeStruct((n, d_hid), jnp.float32),
        ),
    )(x, w1, p0, p1)


def _tc_mid(q0, q1, dinv, self1, b1):
    n, d_hid = q0.shape

    def body(q0_ref, q1_ref, dinv_ref, self1_ref, b1_ref,
             h2pre_ref, self2_ref):
        dinv = dinv_ref[...]
        h = dinv * (q0_ref[...] + q1_ref[...]) + self1_ref[...] + b1_ref[...]
        h = jnp.maximum(h, 0.0)
        h2pre_ref[...] = h * dinv
        self2_ref[...] = h * (dinv * dinv)

    return pl.pallas_call(
        body,
        out_shape=(
            jax.ShapeDtypeStruct((n, d_hid), jnp.float32),
            jax.ShapeDtypeStruct((n, d_hid), jnp.float32),
        ),
    )(q0, q1, dinv, self1, b1)


def _tc_fin(q0, q1, dinv, self2, w2, b2):
    n, d_out = q0.shape[0], w2.shape[1]

    def body(q0_ref, q1_ref, dinv_ref, self2_ref, w2_ref, b2_ref, out_ref):
        agg = dinv_ref[...] * (q0_ref[...] + q1_ref[...]) + self2_ref[...]
        o = jnp.dot(agg, w2_ref[...], preferred_element_type=jnp.float32)
        o = o + b2_ref[...]
        m = jnp.max(o, axis=1, keepdims=True)
        e = jnp.exp(o - m)
        out_ref[...] = (o - m) - jnp.log(jnp.sum(e, axis=1, keepdims=True))

    return pl.pallas_call(
        body,
        out_shape=jax.ShapeDtypeStruct((n, d_out), jnp.float32),
    )(q0, q1, dinv, self2, w2, b2)


def kernel(x, edge_index, W1, b1, W2, b2):
    n, _ = x.shape
    e = edge_index.shape[1]
    d_hid = W1.shape[1]

    n_acc = ((n + 1 + NS * L - 1) // (NS * L)) * (NS * L)
    chunks = (e + NW * CH - 1) // (NW * CH)
    e_pad = NW * chunks * CH

    src = edge_index[0]
    dst = edge_index[1]
    pad = e_pad - e
    src_p = jnp.concatenate(
        [src, jnp.zeros((pad,), jnp.int32)]).reshape(NW, chunks, CH)
    dst_p = jnp.concatenate(
        [dst, jnp.full((pad,), n, jnp.int32)]).reshape(NW, chunks, CH)

    deg_part = _deg_kernel(n_acc, chunks)(dst_p)
    p0 = deg_part[0, :n].reshape(n, 1)
    p1 = deg_part[1, :n].reshape(n, 1)

    dinv, h_pre, self1 = _tc_pre(x, W1, p0, p1)

    agg1 = _agg_kernel(n_acc, chunks, d_hid)(src_p, dst_p, h_pre)
    h2_pre, self2 = _tc_mid(agg1[0, :n], agg1[1, :n], dinv, self1,
                            b1.reshape(1, d_hid))

    agg2 = _agg_kernel(n_acc, chunks, d_hid)(src_p, dst_p, h2_pre)
    return _tc_fin(agg2[0, :n], agg2[1, :n], dinv, self2, W2,
                   b2.reshape(1, W2.shape[1]))


# speedup vs baseline: 34.7837x; 1.6606x over previous
import functools

import jax
import jax.numpy as jnp
from jax import lax
from jax.experimental import pallas as pl
from jax.experimental.pallas import tpu as pltpu
from jax.experimental.pallas import tpu_sc as plsc

NC = 2
NS = 16
NW = NC * NS
CH = 128
L = 16
K = 8


def _sc_mesh():
    return plsc.VectorSubcoreMesh(core_axis_name="c", subcore_axis_name="s")


_SC_PARAMS = pltpu.CompilerParams(use_tc_tiling_on_sc=False)


def _deg_kernel(n_acc, chunks):
    zrows = n_acc // NS

    @functools.partial(
        pl.kernel,
        out_type=jax.ShapeDtypeStruct((NC, n_acc), jnp.float32),
        mesh=_sc_mesh(),
        compiler_params=_SC_PARAMS,
        scratch_types=[
            pltpu.VMEM((chunks, 2, CH), jnp.int32),
            pltpu.VMEM((CH,), jnp.float32),
            pltpu.VMEM((zrows,), jnp.float32),
            pltpu.VMEM_SHARED((n_acc,), jnp.float32),
            pltpu.SemaphoreType.DMA,
        ],
    )
    def deg_kernel(edges_hbm, out_hbm, idxall, ones_v, zbuf, acc, ssem):
        c = lax.axis_index("c")
        s = lax.axis_index("s")
        wid = c * NS + s

        def _fill(i, _):
            zbuf[pl.ds(i * L, L)] = jnp.zeros((L,), jnp.float32)
            ones_v[pl.ds(lax.rem(i, CH // L) * L, L)] = jnp.ones((L,), jnp.float32)
            return 0

        lax.fori_loop(0, zrows // L, _fill, 0)
        pltpu.sync_copy(zbuf, acc.at[pl.ds(s * zrows, zrows)])
        pltpu.sync_copy(edges_hbm.at[wid], idxall)
        plsc.subcore_barrier()

        def _group(g, _):
            for k in range(K):
                pltpu.async_copy(ones_v, acc.at[idxall.at[g * K + k, 1]],
                                 ssem, add=True)
            for k in range(K):
                pltpu.make_async_copy(ones_v, acc.at[idxall.at[g * K + k, 1]],
                                      ssem).wait()
            return 0

        lax.fori_loop(0, chunks // K, _group, 0)
        plsc.subcore_barrier()
        pltpu.sync_copy(acc.at[pl.ds(s * zrows, zrows)],
                        out_hbm.at[c, pl.ds(s * zrows, zrows)])

    return deg_kernel


def _agg_kernel(n_acc, chunks, d):
    zrows = n_acc // NS
    npairs = chunks // 2

    @functools.partial(
        pl.kernel,
        out_type=jax.ShapeDtypeStruct((NC, n_acc, d), jnp.float32),
        mesh=_sc_mesh(),
        compiler_params=_SC_PARAMS,
        scratch_types=[
            pltpu.VMEM((chunks, 2, CH), jnp.int32),
            pltpu.VMEM((CH, d), jnp.float32),
            pltpu.VMEM((CH, d), jnp.float32),
            pltpu.VMEM((zrows, d), jnp.float32),
            pltpu.VMEM_SHARED((n_acc, d), jnp.float32),
            pltpu.SemaphoreType.DMA,
            pltpu.SemaphoreType.DMA,
            pltpu.SemaphoreType.DMA,
            pltpu.SemaphoreType.DMA,
        ],
    )
    def agg_kernel(edges_hbm, table_hbm, out_hbm,
                   idxall, rows_a, rows_b, zbuf, acc,
                   gsem_a, gsem_b, ssem_a, ssem_b):
        c = lax.axis_index("c")
        s = lax.axis_index("s")
        wid = c * NS + s

        def _fill(i, _):
            zbuf[i, :] = jnp.zeros((L,), jnp.float32)
            return 0

        lax.fori_loop(0, zrows, _fill, 0)
        pltpu.sync_copy(zbuf, acc.at[pl.ds(s * zrows, zrows)])
        pltpu.sync_copy(edges_hbm.at[wid], idxall)
        plsc.subcore_barrier()

        pltpu.async_copy(table_hbm.at[idxall.at[0, 0]], rows_a, gsem_a)

        def _pair(p, _):
            a = 2 * p
            b = a + 1
            pltpu.make_async_copy(
                table_hbm.at[idxall.at[a, 0]], rows_a, gsem_a).wait()
            pltpu.async_copy(rows_a, acc.at[idxall.at[a, 1]], ssem_a, add=True)
            @pl.when(p > 0)
            def _():
                pltpu.make_async_copy(
                    rows_b, acc.at[idxall.at[a, 1]], ssem_b).wait()
            pltpu.async_copy(table_hbm.at[idxall.at[b, 0]], rows_b, gsem_b)
            pltpu.make_async_copy(
                rows_a, acc.at[idxall.at[a, 1]], ssem_a).wait()
            @pl.when(p < npairs - 1)
            def _():
                pltpu.async_copy(
                    table_hbm.at[idxall.at[a + 2, 0]], rows_a, gsem_a)
            pltpu.make_async_copy(
                table_hbm.at[idxall.at[b, 0]], rows_b, gsem_b).wait()
            pltpu.async_copy(rows_b, acc.at[idxall.at[b, 1]], ssem_b, add=True)
            return 0

        lax.fori_loop(0, npairs, _pair, 0)
        pltpu.make_async_copy(
            rows_b, acc.at[idxall.at[chunks - 1, 1]], ssem_b).wait()
        plsc.subcore_barrier()
        pltpu.sync_copy(acc.at[pl.ds(s * zrows, zrows)],
                        out_hbm.at[c, pl.ds(s * zrows, zrows)])

    return agg_kernel


def _tc_pre(x, w1, p0, p1):
    n, d_hid = x.shape[0], w1.shape[1]

    def body(x_ref, w1_ref, p0_ref, p1_ref, dinv_ref, hpre_ref, self1_ref):
        dinv = lax.rsqrt(1.0 + p0_ref[...] + p1_ref[...])
        h1 = jnp.dot(x_ref[...], w1_ref[...], preferred_element_type=jnp.float32)
        dinv_ref[...] = dinv
        hpre_ref[...] = h1 * dinv
        self1_ref[...] = h1 * (dinv * dinv)

    return pl.pallas_call(
        body,
        out_shape=(
            jax.ShapeDtypeStruct((n, 1), jnp.float32),
            jax.ShapeDtypeStruct((n, d_hid), jnp.float32),
            jax.ShapeDtypeStruct((n, d_hid), jnp.float32),
        ),
    )(x, w1, p0, p1)


def _tc_mid(q0, q1, dinv, self1, b1):
    n, d_hid = q0.shape

    def body(q0_ref, q1_ref, dinv_ref, self1_ref, b1_ref,
             h2pre_ref, self2_ref):
        dinv = dinv_ref[...]
        h = dinv * (q0_ref[...] + q1_ref[...]) + self1_ref[...] + b1_ref[...]
        h = jnp.maximum(h, 0.0)
        h2pre_ref[...] = h * dinv
        self2_ref[...] = h * (dinv * dinv)

    return pl.pallas_call(
        body,
        out_shape=(
            jax.ShapeDtypeStruct((n, d_hid), jnp.float32),
            jax.ShapeDtypeStruct((n, d_hid), jnp.float32),
        ),
    )(q0, q1, dinv, self1, b1)


def _tc_fin(q0, q1, dinv, self2, w2, b2):
    n, d_out = q0.shape[0], w2.shape[1]

    def body(q0_ref, q1_ref, dinv_ref, self2_ref, w2_ref, b2_ref, out_ref):
        agg = dinv_ref[...] * (q0_ref[...] + q1_ref[...]) + self2_ref[...]
        o = jnp.dot(agg, w2_ref[...], preferred_element_type=jnp.float32)
        o = o + b2_ref[...]
        m = jnp.max(o, axis=1, keepdims=True)
        e = jnp.exp(o - m)
        out_ref[...] = (o - m) - jnp.log(jnp.sum(e, axis=1, keepdims=True))

    return pl.pallas_call(
        body,
        out_shape=jax.ShapeDtypeStruct((n, d_out), jnp.float32),
    )(q0, q1, dinv, self2, w2, b2)


def kernel(x, edge_index, W1, b1, W2, b2):
    n, _ = x.shape
    e = edge_index.shape[1]
    d_hid = W1.shape[1]

    n_acc = ((n + 1 + NS * L - 1) // (NS * L)) * (NS * L)
    chunks = -(-e // (NW * CH))
    chunks = ((chunks + K - 1) // K) * K
    e_pad = NW * chunks * CH

    src = edge_index[0]
    dst = edge_index[1]
    pad = e_pad - e
    src_p = jnp.concatenate(
        [src, jnp.zeros((pad,), jnp.int32)]).reshape(NW, chunks, CH)
    dst_p = jnp.concatenate(
        [dst, jnp.full((pad,), n, jnp.int32)]).reshape(NW, chunks, CH)
    edges_p = jnp.stack([src_p, dst_p], axis=2)

    deg_part = _deg_kernel(n_acc, chunks)(edges_p)
    p0 = deg_part[0, :n].reshape(n, 1)
    p1 = deg_part[1, :n].reshape(n, 1)

    dinv, h_pre, self1 = _tc_pre(x, W1, p0, p1)

    agg1 = _agg_kernel(n_acc, chunks, d_hid)(edges_p, h_pre)
    h2_pre, self2 = _tc_mid(agg1[0, :n], agg1[1, :n], dinv, self1,
                            b1.reshape(1, d_hid))

    agg2 = _agg_kernel(n_acc, chunks, d_hid)(edges_p, h2_pre)
    return _tc_fin(agg2[0, :n], agg2[1, :n], dinv, self2, W2,
                   b2.reshape(1, W2.shape[1]))


# speedup vs baseline: 58.4337x; 1.6799x over previous
import functools

import jax
import jax.numpy as jnp
from jax import lax
from jax.experimental import pallas as pl
from jax.experimental.pallas import tpu as pltpu
from jax.experimental.pallas import tpu_sc as plsc

NC = 2
NS = 16
NW = NC * NS
CH = 80
L = 16
NB = 8
GA = 4


def _sc_mesh():
    return plsc.VectorSubcoreMesh(core_axis_name="c", subcore_axis_name="s")


_SC_PARAMS = pltpu.CompilerParams(use_tc_tiling_on_sc=False)


def _deg_kernel(n_acc, cm):
    zrows = n_acc // NS
    G, rem = cm // NB, cm % NB

    @functools.partial(
        pl.kernel,
        out_type=jax.ShapeDtypeStruct((NC, n_acc), jnp.float32),
        mesh=_sc_mesh(),
        compiler_params=_SC_PARAMS,
        scratch_types=[
            pltpu.VMEM((cm, CH), jnp.int32),
            pltpu.VMEM((CH,), jnp.float32),
            pltpu.VMEM((zrows,), jnp.float32),
            pltpu.VMEM_SHARED((n_acc,), jnp.float32),
        ] + [pltpu.SemaphoreType.DMA] * NB,
    )
    def deg_kernel(dstm_hbm, out_hbm, didx, ones_v, zbuf, acc,
                   s0, s1, s2, s3, s4, s5, s6, s7):
        ssem = (s0, s1, s2, s3, s4, s5, s6, s7)
        c = lax.axis_index("c")
        s = lax.axis_index("s")
        wid = c * NS + s

        def _fill(i, _):
            zbuf[pl.ds(i * L, L)] = jnp.zeros((L,), jnp.float32)
            ones_v[pl.ds(lax.rem(i, CH // L) * L, L)] = (
                jnp.ones((L,), jnp.float32))
            return 0

        lax.fori_loop(0, zrows // L, _fill, 0)
        pltpu.sync_copy(zbuf, acc.at[pl.ds(s * zrows, zrows)])
        pltpu.sync_copy(dstm_hbm.at[wid], didx)
        plsc.subcore_barrier()

        def _start(i, b):
            pltpu.async_copy(ones_v, acc.at[didx.at[i]], ssem[b], add=True)

        def _wait(b):
            pltpu.make_async_copy(ones_v, acc.at[didx.at[0]], ssem[b]).wait()

        def _group(g, _):
            for b in range(NB):
                @pl.when(g > 0)
                def _():
                    _wait(b)
                _start(g * NB + b, b)
            return 0

        lax.fori_loop(0, G, _group, 0)
        for i in range(G * NB, cm):
            if i >= NB:
                _wait(i % NB)
            _start(i, i % NB)
        for j in range(max(0, cm - NB), cm):
            _wait(j % NB)
        plsc.subcore_barrier()
        pltpu.sync_copy(acc.at[pl.ds(s * zrows, zrows)],
                        out_hbm.at[c, pl.ds(s * zrows, zrows)])

    return deg_kernel


def _agg_kernel(n_acc, cm, d):
    zrows = n_acc // NS
    G, rem = cm // NB, cm % NB
    assert rem >= GA and cm >= NB

    @functools.partial(
        pl.kernel,
        out_type=jax.ShapeDtypeStruct((NC, n_acc, d), jnp.float32),
        mesh=_sc_mesh(),
        compiler_params=_SC_PARAMS,
        scratch_types=[
            pltpu.VMEM((cm, CH), jnp.int32),
            pltpu.VMEM((cm, CH), jnp.int32),
        ] + [pltpu.VMEM((CH, d), jnp.float32)] * NB + [
            pltpu.VMEM((zrows, d), jnp.float32),
            pltpu.VMEM_SHARED((n_acc, d), jnp.float32),
        ] + [pltpu.SemaphoreType.DMA] * (2 * NB),
    )
    def agg_kernel(srcm_hbm, dstm_hbm, table_hbm, out_hbm,
                   sidx, didx, r0, r1, r2, r3, r4, r5, r6, r7, zbuf, acc,
                   g0, g1, g2, g3, g4, g5, g6, g7,
                   t0, t1, t2, t3, t4, t5, t6, t7):
        rows = (r0, r1, r2, r3, r4, r5, r6, r7)
        gsem = (g0, g1, g2, g3, g4, g5, g6, g7)
        ssem = (t0, t1, t2, t3, t4, t5, t6, t7)
        c = lax.axis_index("c")
        s = lax.axis_index("s")
        wid = c * NS + s

        def _fill(i, _):
            zbuf[i, :] = jnp.zeros((L,), jnp.float32)
            return 0

        lax.fori_loop(0, zrows, _fill, 0)
        pltpu.sync_copy(zbuf, acc.at[pl.ds(s * zrows, zrows)])
        pltpu.sync_copy(srcm_hbm.at[wid], sidx)
        pltpu.sync_copy(dstm_hbm.at[wid], didx)
        plsc.subcore_barrier()

        def _gather(i, b):
            pltpu.async_copy(table_hbm.at[sidx.at[i]], rows[b], gsem[b])

        def _gwait(b):
            pltpu.make_async_copy(
                table_hbm.at[sidx.at[0]], rows[b], gsem[b]).wait()

        def _scatter(i, b):
            pltpu.async_copy(rows[b], acc.at[didx.at[i]], ssem[b], add=True)

        def _swait(b):
            pltpu.make_async_copy(rows[b], acc.at[didx.at[0]], ssem[b]).wait()

        for j in range(GA):
            _gather(j, j)

        def _group(g, _):
            for b in range(NB):
                i = g * NB + b
                tb = (b + GA) % NB
                if b + GA >= NB:
                    _swait(tb)
                    _gather(i + GA, tb)
                else:
                    @pl.when(g > 0)
                    def _():
                        _swait(tb)
                    _gather(i + GA, tb)
                _gwait(b)
                _scatter(i, b)
            return 0

        lax.fori_loop(0, G, _group, 0)
        for i in range(G * NB, cm):
            b = i % NB
            tb = (b + GA) % NB
            if i + GA < cm:
                _swait(tb)
                _gather(i + GA, tb)
            _gwait(b)
            _scatter(i, b)
        for j in range(max(0, cm - NB), cm):
            _swait(j % NB)
        plsc.subcore_barrier()
        pltpu.sync_copy(acc.at[pl.ds(s * zrows, zrows)],
                        out_hbm.at[c, pl.ds(s * zrows, zrows)])

    return agg_kernel


def _tc_mm(x, w1):
    n, d_hid = x.shape[0], w1.shape[1]

    def body(x_ref, w1_ref, h1_ref):
        h1_ref[...] = jnp.dot(x_ref[...], w1_ref[...],
                              preferred_element_type=jnp.float32)

    return pl.pallas_call(
        body,
        out_shape=jax.ShapeDtypeStruct((n, d_hid), jnp.float32),
    )(x, w1)


def _tc_pre(h1, deg_part):
    n, d_hid = h1.shape

    def body(h1_ref, deg_ref, dinv_ref, hpre_ref, self1_ref):
        dinv = lax.rsqrt(1.0 + deg_ref[0] + deg_ref[1])
        h1 = h1_ref[...]
        dinv_ref[...] = dinv
        hpre_ref[...] = h1 * dinv
        self1_ref[...] = h1 * (dinv * dinv)

    return pl.pallas_call(
        body,
        out_shape=(
            jax.ShapeDtypeStruct((n, 1), jnp.float32),
            jax.ShapeDtypeStruct((n, d_hid), jnp.float32),
            jax.ShapeDtypeStruct((n, d_hid), jnp.float32),
        ),
    )(h1, deg_part)


def _tc_mid(agg1, dinv, self1, b1):
    n, d_hid = self1.shape

    def body(agg_ref, dinv_ref, self1_ref, b1_ref, h2pre_ref, self2_ref):
        dinv = dinv_ref[...]
        h = (dinv * (agg_ref[0, :n, :] + agg_ref[1, :n, :])
             + self1_ref[...] + b1_ref[...])
        h = jnp.maximum(h, 0.0)
        h2pre_ref[...] = h * dinv
        self2_ref[...] = h * (dinv * dinv)

    return pl.pallas_call(
        body,
        out_shape=(
            jax.ShapeDtypeStruct((n, d_hid), jnp.float32),
            jax.ShapeDtypeStruct((n, d_hid), jnp.float32),
        ),
    )(agg1, dinv, self1, b1)


def _tc_fin(agg2, dinv, self2, w2, b2):
    n = self2.shape[0]
    d_out = w2.shape[1]

    def body(agg_ref, dinv_ref, self2_ref, w2_ref, b2_ref, out_ref):
        agg = (dinv_ref[...] * (agg_ref[0, :n, :] + agg_ref[1, :n, :])
               + self2_ref[...])
        o = jnp.dot(agg, w2_ref[...], preferred_element_type=jnp.float32)
        o = o + b2_ref[...]
        m = jnp.max(o, axis=1, keepdims=True)
        e = jnp.exp(o - m)
        out_ref[...] = (o - m) - jnp.log(jnp.sum(e, axis=1, keepdims=True))

    return pl.pallas_call(
        body,
        out_shape=jax.ShapeDtypeStruct((n, d_out), jnp.float32),
    )(agg2, dinv, self2, w2, b2)


def kernel(x, edge_index, W1, b1, W2, b2):
    n, _ = x.shape
    e = edge_index.shape[1]
    d_hid = W1.shape[1]

    n_acc = ((n + 1 + NS * L - 1) // (NS * L)) * (NS * L)
    e_pad = ((e + NW * CH - 1) // (NW * CH)) * (NW * CH)
    cm = e_pad // (NW * CH)

    src = edge_index[0]
    dst = edge_index[1]
    if e_pad != e:
        src = jnp.concatenate([src, jnp.zeros((e_pad - e,), jnp.int32)])
        dst = jnp.concatenate([dst, jnp.full((e_pad - e,), n, jnp.int32)])
    srcm = src.reshape(NW, cm, CH)
    dstm = dst.reshape(NW, cm, CH)

    h1 = _tc_mm(x, W1)
    deg_part = _deg_kernel(n_acc, cm)(dstm)
    deg_col = deg_part[:, :n].reshape(2, n, 1)
    dinv, h_pre, self1 = _tc_pre(h1, deg_col)

    agg1 = _agg_kernel(n_acc, cm, d_hid)(srcm, dstm, h_pre)
    h2_pre, self2 = _tc_mid(agg1, dinv, self1, b1.reshape(1, d_hid))

    agg2 = _agg_kernel(n_acc, cm, d_hid)(srcm, dstm, h2_pre)
    return _tc_fin(agg2, dinv, self2, W2, b2.reshape(1, W2.shape[1]))
